# baseline (device time: 193685 ns/iter reference)
import jax
import jax.numpy as jnp
from jax import lax
from jax.experimental import pallas as pl
from jax.experimental.pallas import tpu as pltpu

N_DEV = 32
P = 8
Z = 4
M = 2048
N = 2048
SC = M // P
CH = M // N_DEV
HC = N // 2


def kernel(x, w_mat):
    def body(x_ref, w_ref, out_ref, acc_ref, c1_ref, c2_ref,
             send1, recv1, send2, recv2):
        my = lax.axis_index("i")
        g = lax.rem(my, P)
        plane0 = my - g
        z = lax.div(my, P)

        nbrs = [
            plane0 + lax.rem(g + 1, P),
            plane0 + lax.rem(g + P - 1, P),
            lax.rem(my + P, N_DEV),
            lax.rem(my + N_DEV - P, N_DEV),
        ]

        barrier_sem = pltpu.get_barrier_semaphore()
        for nbr in nbrs:
            pl.semaphore_signal(
                barrier_sem, inc=1,
                device_id=(nbr,), device_id_type=pl.DeviceIdType.MESH,
            )
        pl.semaphore_wait(barrier_sem, len(nbrs))

        acc_ref[:, :] = jnp.dot(
            x_ref[:, :], w_ref[:, :], preferred_element_type=jnp.float32
        )

        def cols(d):
            return pl.ds(d * HC, HC)

        def p1_send_sc(d, s):
            return lax.rem(g + (P - 1 - s), P) if d == 0 else lax.rem(g + s + 1, P)

        def p1_recv_sc(d, s):
            return lax.rem(g + (P - 2 - s), P) if d == 0 else lax.rem(g + s + 2, P)

        def p1_start(d, s):
            rdma = pltpu.make_async_remote_copy(
                src_ref=acc_ref.at[pl.ds(p1_send_sc(d, s) * SC, SC), cols(d)],
                dst_ref=c1_ref.at[d, s],
                send_sem=send1.at[d, s],
                recv_sem=recv1.at[d, s],
                device_id=(nbrs[d],),
                device_id_type=pl.DeviceIdType.MESH,
            )
            rdma.start()
            return rdma

        def p2_send_zc(d, s):
            return lax.rem(z + (Z - 1 - s), Z) if d == 0 else lax.rem(z + s + 1, Z)

        def p2_recv_zc(d, s):
            return lax.rem(z + (Z - 2 - s), Z) if d == 0 else lax.rem(z + s + 2, Z)

        def p2_start(d, s):
            rows = pl.ds(g * SC + p2_send_zc(d, s) * CH, CH)
            rdma = pltpu.make_async_remote_copy(
                src_ref=acc_ref.at[rows, cols(d)],
                dst_ref=c2_ref.at[d, s],
                send_sem=send2.at[d, s],
                recv_sem=recv2.at[d, s],
                device_id=(nbrs[2 + d],),
                device_id_type=pl.DeviceIdType.MESH,
            )
            rdma.start()
            return rdma

        p1 = {(d, 0): p1_start(d, 0) for d in (0, 1)}
        p2 = {}
        for s in range(P - 1):
            for d in (0, 1):
                p1[(d, s)].wait_recv()
                rows = pl.ds(p1_recv_sc(d, s) * SC, SC)
                acc_ref[rows, cols(d)] = (
                    acc_ref[rows, cols(d)] + c1_ref[d, s, :, :]
                )
                if s < P - 2:
                    p1[(d, s + 1)] = p1_start(d, s + 1)
                else:
                    p2[(d, 0)] = p2_start(d, 0)

        for s in range(Z - 1):
            for d in (0, 1):
                p2[(d, s)].wait_recv()
                rows = pl.ds(g * SC + p2_recv_zc(d, s) * CH, CH)
                acc_ref[rows, cols(d)] = (
                    acc_ref[rows, cols(d)] + c2_ref[d, s, :, :]
                )
                if s < Z - 2:
                    p2[(d, s + 1)] = p2_start(d, s + 1)

        rdmas = list(p1.values()) + list(p2.values())

        for rdma in rdmas:
            rdma.wait_send()

        out_ref[:, :] = acc_ref[pl.ds(my * CH, CH), :]

    return pl.pallas_call(
        body,
        out_shape=jax.ShapeDtypeStruct((CH, N), jnp.float32),
        in_specs=[
            pl.BlockSpec(memory_space=pltpu.VMEM),
            pl.BlockSpec(memory_space=pltpu.VMEM),
        ],
        out_specs=pl.BlockSpec(memory_space=pltpu.VMEM),
        scratch_shapes=[
            pltpu.VMEM((M, N), jnp.float32),
            pltpu.VMEM((2, P - 1, SC, HC), jnp.float32),
            pltpu.VMEM((2, Z - 1, CH, HC), jnp.float32),
            pltpu.SemaphoreType.DMA((2, P - 1)),
            pltpu.SemaphoreType.DMA((2, P - 1)),
            pltpu.SemaphoreType.DMA((2, Z - 1)),
            pltpu.SemaphoreType.DMA((2, Z - 1)),
        ],
        compiler_params=pltpu.CompilerParams(collective_id=0),
    )(x, w_mat)


# device time: 128051 ns/iter; 1.5126x vs baseline; 1.5126x over previous
import jax
import jax.numpy as jnp
import numpy as np
from jax import lax
from jax.experimental import pallas as pl
from jax.experimental.pallas import tpu as pltpu

N_DEV = 32
P = 8
Z = 4
M = 2048
N = 2048
SC = M // P
CH = M // N_DEV
HC = N // 2

_n = np.arange(M)
ROW_PERM = 512 * ((_n % 256) // 64) + 64 * (_n // 256) + (_n % 64)


def kernel(x, w_mat):
    def body(x_ref, w_ref, out_ref, acc_ref, c1_ref, c2_ref,
             send1, recv1, send2, recv2):
        my = lax.axis_index("i")
        g = lax.rem(my, P)
        plane0 = my - g
        z = lax.div(my, P)

        PI = (0, 1, 2, 5, 6, 7, 4, 3)
        INV = (0, 1, 2, 7, 6, 3, 4, 5)
        NEXT = (1, 2, 5, 0, 3, 6, 7, 4)
        PREV = (3, 0, 1, 4, 7, 2, 5, 6)

        def lut(table, idx):
            v = jnp.int32(table[0])
            for k in range(1, len(table)):
                v = jnp.where(idx == k, jnp.int32(table[k]), v)
            return v

        q = lut(INV, g)

        nbrs = [
            plane0 + lut(NEXT, g),
            plane0 + lut(PREV, g),
            lax.rem(my + P, N_DEV),
            lax.rem(my + N_DEV - P, N_DEV),
        ]

        barrier_sem = pltpu.get_barrier_semaphore()
        for nbr in nbrs:
            pl.semaphore_signal(
                barrier_sem, inc=1,
                device_id=(nbr,), device_id_type=pl.DeviceIdType.MESH,
            )
        pl.semaphore_wait(barrier_sem, len(nbrs))

        acc_ref[:, :] = jnp.dot(
            x_ref[:, :], w_ref[:, :], preferred_element_type=jnp.float32
        )

        def cols(d):
            return pl.ds(d * HC, HC)

        def p1_send_sc(d, s):
            pos = (q + (P - 1 - s)) if d == 0 else (q + s + 1)
            return lut(PI, lax.rem(pos, P))

        def p1_recv_sc(d, s):
            pos = (q + (P - 2 - s)) if d == 0 else (q + s + 2)
            return lut(PI, lax.rem(pos, P))

        def p1_start(d, s):
            rdma = pltpu.make_async_remote_copy(
                src_ref=acc_ref.at[pl.ds(p1_send_sc(d, s) * SC, SC), cols(d)],
                dst_ref=c1_ref.at[d, s],
                send_sem=send1.at[d, s],
                recv_sem=recv1.at[d, s],
                device_id=(nbrs[d],),
                device_id_type=pl.DeviceIdType.MESH,
            )
            rdma.start()
            return rdma

        def p2_send_zc(d, s):
            return lax.rem(z + (Z - 1 - s), Z) if d == 0 else lax.rem(z + s + 1, Z)

        def p2_recv_zc(d, s):
            return lax.rem(z + (Z - 2 - s), Z) if d == 0 else lax.rem(z + s + 2, Z)

        def p2_start(d, s):
            rows = pl.ds(g * SC + p2_send_zc(d, s) * CH, CH)
            rdma = pltpu.make_async_remote_copy(
                src_ref=acc_ref.at[rows, cols(d)],
                dst_ref=c2_ref.at[d, s],
                send_sem=send2.at[d, s],
                recv_sem=recv2.at[d, s],
                device_id=(nbrs[2 + d],),
                device_id_type=pl.DeviceIdType.MESH,
            )
            rdma.start()
            return rdma

        p1 = {(d, 0): p1_start(d, 0) for d in (0, 1)}
        p2 = {}
        for s in range(P - 1):
            for d in (0, 1):
                p1[(d, s)].wait_recv()
                rows = pl.ds(p1_recv_sc(d, s) * SC, SC)
                acc_ref[rows, cols(d)] = (
                    acc_ref[rows, cols(d)] + c1_ref[d, s, :, :]
                )
                if s < P - 2:
                    p1[(d, s + 1)] = p1_start(d, s + 1)
                else:
                    p2[(d, 0)] = p2_start(d, 0)

        for s in range(Z - 1):
            for d in (0, 1):
                p2[(d, s)].wait_recv()
                rows = pl.ds(g * SC + p2_recv_zc(d, s) * CH, CH)
                acc_ref[rows, cols(d)] = (
                    acc_ref[rows, cols(d)] + c2_ref[d, s, :, :]
                )
                if s < Z - 2:
                    p2[(d, s + 1)] = p2_start(d, s + 1)

        rdmas = list(p1.values()) + list(p2.values())

        for rdma in rdmas:
            rdma.wait_send()

        out_ref[:, :] = acc_ref[pl.ds(g * SC + z * CH, CH), :]

    return pl.pallas_call(
        body,
        out_shape=jax.ShapeDtypeStruct((CH, N), jnp.float32),
        in_specs=[
            pl.BlockSpec(memory_space=pltpu.VMEM),
            pl.BlockSpec(memory_space=pltpu.VMEM),
        ],
        out_specs=pl.BlockSpec(memory_space=pltpu.VMEM),
        scratch_shapes=[
            pltpu.VMEM((M, N), jnp.float32),
            pltpu.VMEM((2, P - 1, SC, HC), jnp.float32),
            pltpu.VMEM((2, Z - 1, CH, HC), jnp.float32),
            pltpu.SemaphoreType.DMA((2, P - 1)),
            pltpu.SemaphoreType.DMA((2, P - 1)),
            pltpu.SemaphoreType.DMA((2, Z - 1)),
            pltpu.SemaphoreType.DMA((2, Z - 1)),
        ],
        compiler_params=pltpu.CompilerParams(collective_id=0),
    )(jnp.take(x, jnp.asarray(ROW_PERM), axis=0), w_mat)


# device time: 125203 ns/iter; 1.5470x vs baseline; 1.0227x over previous
import jax
import jax.numpy as jnp
import numpy as np
from jax import lax
from jax.experimental import pallas as pl
from jax.experimental.pallas import tpu as pltpu

N_DEV = 32
P = 8
Z = 4
M = 2048
N = 2048
SC = M // P
CH = M // N_DEV
HC = N // 2


def kernel(x, w_mat):
    def body(x_ref, w_ref, out_ref, xp_ref, acc_ref, c1_ref, c2_ref,
             send1, recv1, send2, recv2):
        my = lax.axis_index("i")
        g = lax.rem(my, P)
        plane0 = my - g
        z = lax.div(my, P)

        PI = (0, 1, 2, 5, 6, 7, 4, 3)
        INV = (0, 1, 2, 7, 6, 3, 4, 5)
        NEXT = (1, 2, 5, 0, 3, 6, 7, 4)
        PREV = (3, 0, 1, 4, 7, 2, 5, 6)

        def lut(table, idx):
            v = jnp.int32(table[0])
            for k in range(1, len(table)):
                v = jnp.where(idx == k, jnp.int32(table[k]), v)
            return v

        q = lut(INV, g)

        nbrs = [
            plane0 + lut(NEXT, g),
            plane0 + lut(PREV, g),
            lax.rem(my + P, N_DEV),
            lax.rem(my + N_DEV - P, N_DEV),
        ]

        barrier_sem = pltpu.get_barrier_semaphore()
        for nbr in nbrs:
            pl.semaphore_signal(
                barrier_sem, inc=1,
                device_id=(nbr,), device_id_type=pl.DeviceIdType.MESH,
            )
        pl.semaphore_wait(barrier_sem, len(nbrs))

        for gb in range(P):
            for zb in range(Z):
                xp_ref[pl.ds(gb * SC + zb * CH, CH), :] = (
                    x_ref[pl.ds(zb * (P * CH) + gb * CH, CH), :]
                )
        acc_ref[:, :] = jnp.dot(
            xp_ref[:, :], w_ref[:, :], preferred_element_type=jnp.float32
        )

        def cols(d):
            return pl.ds(d * HC, HC)

        def p1_send_sc(d, s):
            pos = (q + (P - 1 - s)) if d == 0 else (q + s + 1)
            return lut(PI, lax.rem(pos, P))

        def p1_recv_sc(d, s):
            pos = (q + (P - 2 - s)) if d == 0 else (q + s + 2)
            return lut(PI, lax.rem(pos, P))

        def p1_start(d, s):
            rdma = pltpu.make_async_remote_copy(
                src_ref=acc_ref.at[pl.ds(p1_send_sc(d, s) * SC, SC), cols(d)],
                dst_ref=c1_ref.at[d, s],
                send_sem=send1.at[d, s],
                recv_sem=recv1.at[d, s],
                device_id=(nbrs[d],),
                device_id_type=pl.DeviceIdType.MESH,
            )
            rdma.start()
            return rdma

        def p2_send_zc(d, s):
            return lax.rem(z + (Z - 1 - s), Z) if d == 0 else lax.rem(z + s + 1, Z)

        def p2_recv_zc(d, s):
            return lax.rem(z + (Z - 2 - s), Z) if d == 0 else lax.rem(z + s + 2, Z)

        def p2_start(d, s):
            rows = pl.ds(g * SC + p2_send_zc(d, s) * CH, CH)
            rdma = pltpu.make_async_remote_copy(
                src_ref=acc_ref.at[rows, cols(d)],
                dst_ref=c2_ref.at[d, s],
                send_sem=send2.at[d, s],
                recv_sem=recv2.at[d, s],
                device_id=(nbrs[2 + d],),
                device_id_type=pl.DeviceIdType.MESH,
            )
            rdma.start()
            return rdma

        p1 = {(d, 0): p1_start(d, 0) for d in (0, 1)}
        p2 = {}
        for s in range(P - 1):
            for d in (0, 1):
                p1[(d, s)].wait_recv()
                rows = pl.ds(p1_recv_sc(d, s) * SC, SC)
                acc_ref[rows, cols(d)] = (
                    acc_ref[rows, cols(d)] + c1_ref[d, s, :, :]
                )
                if s < P - 2:
                    p1[(d, s + 1)] = p1_start(d, s + 1)
                else:
                    p2[(d, 0)] = p2_start(d, 0)

        for s in range(Z - 1):
            for d in (0, 1):
                p2[(d, s)].wait_recv()
                rows = pl.ds(g * SC + p2_recv_zc(d, s) * CH, CH)
                acc_ref[rows, cols(d)] = (
                    acc_ref[rows, cols(d)] + c2_ref[d, s, :, :]
                )
                if s < Z - 2:
                    p2[(d, s + 1)] = p2_start(d, s + 1)

        rdmas = list(p1.values()) + list(p2.values())

        for rdma in rdmas:
            rdma.wait_send()

        out_ref[:, :] = acc_ref[pl.ds(g * SC + z * CH, CH), :]

    return pl.pallas_call(
        body,
        out_shape=jax.ShapeDtypeStruct((CH, N), jnp.float32),
        in_specs=[
            pl.BlockSpec(memory_space=pltpu.VMEM),
            pl.BlockSpec(memory_space=pltpu.VMEM),
        ],
        out_specs=pl.BlockSpec(memory_space=pltpu.VMEM),
        scratch_shapes=[
            pltpu.VMEM((M, M // N_DEV), jnp.float32),
            pltpu.VMEM((M, N), jnp.float32),
            pltpu.VMEM((2, P - 1, SC, HC), jnp.float32),
            pltpu.VMEM((2, Z - 1, CH, HC), jnp.float32),
            pltpu.SemaphoreType.DMA((2, P - 1)),
            pltpu.SemaphoreType.DMA((2, P - 1)),
            pltpu.SemaphoreType.DMA((2, Z - 1)),
            pltpu.SemaphoreType.DMA((2, Z - 1)),
        ],
        compiler_params=pltpu.CompilerParams(
            collective_id=0, vmem_limit_bytes=64 * 1024 * 1024
        ),
    )(x, w_mat)


# device time: 96360 ns/iter; 2.0100x vs baseline; 1.2993x over previous
import jax
import jax.numpy as jnp
from jax import lax
from jax.experimental import pallas as pl
from jax.experimental.pallas import tpu as pltpu

N_DEV = 32
P = 8
Z = 4
M = 2048
N = 2048
SC = M // P
CH = M // N_DEV
ZB = M // Z

AC = 1280
BC = N - AC
AH = AC // 2
BH = BC // 2


def kernel(x, w_mat):
    def body(x_ref, w_ref, out_ref, xp_ref, accA_ref, accB_ref,
             cA1, cB1, cA2, cB2,
             sA1, rA1, sB1, rB1, sA2, rA2, sB2, rB2):
        my = lax.axis_index("i")
        g = lax.rem(my, P)
        plane0 = my - g
        z = lax.div(my, P)

        PI = (0, 1, 2, 5, 6, 7, 4, 3)
        INV = (0, 1, 2, 7, 6, 3, 4, 5)
        NEXT = (1, 2, 5, 0, 3, 6, 7, 4)
        PREV = (3, 0, 1, 4, 7, 2, 5, 6)

        def lut(table, idx):
            v = jnp.int32(table[0])
            for k in range(1, len(table)):
                v = jnp.where(idx == k, jnp.int32(table[k]), v)
            return v

        q = lut(INV, g)

        nbrs = [
            plane0 + lut(NEXT, g),
            plane0 + lut(PREV, g),
            lax.rem(my + P, N_DEV),
            lax.rem(my + N_DEV - P, N_DEV),
        ]

        barrier_sem = pltpu.get_barrier_semaphore()
        for nbr in nbrs:
            pl.semaphore_signal(
                barrier_sem, inc=1,
                device_id=(nbr,), device_id_type=pl.DeviceIdType.MESH,
            )
        pl.semaphore_wait(barrier_sem, len(nbrs))

        for gb in range(P):
            for zb in range(Z):
                xp_ref[pl.ds(gb * SC + zb * CH, CH), :] = (
                    x_ref[pl.ds(zb * ZB + gb * CH, CH), :]
                )

        def gemmA(pos):
            rows = pl.ds(lut(PI, lax.rem(pos + P, P)) * SC, SC)
            accA_ref[rows, :] = jnp.dot(
                xp_ref[rows, :], w_ref[:, :AC],
                preferred_element_type=jnp.float32,
            )

        def gemmB(zeta):
            rows = pl.ds(lax.rem(zeta + Z, Z) * ZB, ZB)
            accB_ref[rows, :] = jnp.dot(
                x_ref[rows, :], w_ref[:, AC:],
                preferred_element_type=jnp.float32,
            )

        def ip_send(s):
            return (lax.rem(q + (P - 1 - s), P), lax.rem(q + s + 1, P))

        def ip_recv(s):
            return (lax.rem(q + (P - 2 - s), P), lax.rem(q + s + 2, P))

        def z_send(s):
            return (lax.rem(z + (Z - 1 - s), Z), lax.rem(z + s + 1, Z))

        def z_recv(s):
            return (lax.rem(z + (Z - 2 - s), Z), lax.rem(z + s + 2, Z))

        def start(src, comm, ssem, rsem, d, s, to):
            rdma = pltpu.make_async_remote_copy(
                src_ref=src,
                dst_ref=comm.at[d, s],
                send_sem=ssem.at[d, s],
                recv_sem=rsem.at[d, s],
                device_id=(to,),
                device_id_type=pl.DeviceIdType.MESH,
            )
            rdma.start()
            return rdma

        def a1_start(d, s):
            rows = pl.ds(lut(PI, ip_send(s)[d]) * SC, SC)
            src = accA_ref.at[rows, pl.ds(d * AH, AH)]
            return start(src, cA1, sA1, rA1, d, s, nbrs[d])

        def a2_start(d, s):
            rows = pl.ds(g * SC + z_send(s)[d] * CH, CH)
            src = accA_ref.at[rows, pl.ds(d * AH, AH)]
            return start(src, cA2, sA2, rA2, d, s, nbrs[2 + d])

        def b1_start(d, s):
            rows = pl.ds(z_send(s)[d] * ZB, ZB)
            src = accB_ref.at[rows, pl.ds(d * BH, BH)]
            return start(src, cB1, sB1, rB1, d, s, nbrs[2 + d])

        GX = (0, 1, 1, 0, 0, 1, 1, 0)
        GY = (0, 0, 1, 1, 2, 2, 3, 3)
        PG1 = (1, 0, 3, 2, 5, 4, 7, 6)
        PG2 = (3, 2, 1, 0, 7, 6, 5, 4)
        PG3 = (4, 5, 6, 7, 0, 1, 2, 3)
        S0, S1 = (0, 3, 4, 7), (1, 2, 5, 6)
        T = (((0, 4), (3, 7)), ((1, 5), (2, 6)))
        gx = lut(GX, g)
        gyp = lax.rem(lut(GY, g), 2)

        def pick(m, tab0, tab1):
            return jnp.where(gx == 0, jnp.int32(tab0[m]), jnp.int32(tab1[m]))

        def pick2(j, parity):
            return jnp.where(
                gx == 0,
                jnp.where(parity == 0, jnp.int32(T[0][0][j]), jnp.int32(T[0][1][j])),
                jnp.where(parity == 0, jnp.int32(T[1][0][j]), jnp.int32(T[1][1][j])),
            )

        def b2_chunk(m, recv):
            if m < 4:
                return pick(m, S0, S1) if recv else pick(m, S1, S0)
            if m < 6:
                par = gyp if recv else 1 - gyp
                return pick2(m - 4, par)
            return g if recv else lut(PG3, g)

        def b2_partner(m):
            tab = PG1 if m < 4 else (PG2 if m < 6 else PG3)
            return plane0 + lut(tab, g)

        def b2_start(d, m):
            rows = pl.ds(z * ZB + b2_chunk(m, recv=False) * CH, CH)
            src = accB_ref.at[rows, pl.ds(d * BH, BH)]
            return start(src, cB2, sB2, rB2, d, m, b2_partner(m))

        def b2_acc(d, m):
            B2[(d, m)].wait_recv()
            rows = pl.ds(z * ZB + b2_chunk(m, recv=True) * CH, CH)
            colsd = pl.ds(d * BH, BH)
            accB_ref[rows, colsd] = accB_ref[rows, colsd] + cB2[d, m, :, :]

        gemmB(z + 3)
        gemmB(z + 1)
        B1 = {(d, 0): b1_start(d, 0) for d in (0, 1)}
        gemmA(q - 1)
        gemmA(q + 1)
        A1 = {(d, 0): a1_start(d, 0) for d in (0, 1)}
        gemmB(z + 2)
        gemmB(z)
        gemmA(q - 2)
        gemmA(q + 2)
        gemmA(q - 3)
        gemmA(q + 3)
        gemmA(q + 4)
        gemmA(q)

        A2 = {}
        B2 = {}

        for s in range(P - 1):
            for d in (0, 1):
                A1[(d, s)].wait_recv()
                rows = pl.ds(lut(PI, ip_recv(s)[d]) * SC, SC)
                colsd = pl.ds(d * AH, AH)
                accA_ref[rows, colsd] = (
                    accA_ref[rows, colsd] + cA1[d, s, :, :]
                )
                if s < P - 2:
                    A1[(d, s + 1)] = a1_start(d, s + 1)
                else:
                    A2[(d, 0)] = a2_start(d, 0)
            if s in (2, 4, 6):
                k = s // 2 - 1
                for d in (0, 1):
                    B1[(d, k)].wait_recv()
                    rows = pl.ds(z_recv(k)[d] * ZB, ZB)
                    colsd = pl.ds(d * BH, BH)
                    accB_ref[rows, colsd] = (
                        accB_ref[rows, colsd] + cB1[d, k, :, :]
                    )
                    if k < Z - 2:
                        B1[(d, k + 1)] = b1_start(d, k + 1)
                    else:
                        for m in range(4):
                            B2[(d, m)] = b2_start(d, m)

        def a2_step(k):
            for d in (0, 1):
                A2[(d, k)].wait_recv()
                rows = pl.ds(g * SC + z_recv(k)[d] * CH, CH)
                colsd = pl.ds(d * AH, AH)
                accA_ref[rows, colsd] = (
                    accA_ref[rows, colsd] + cA2[d, k, :, :]
                )
                if k < Z - 2:
                    A2[(d, k + 1)] = a2_start(d, k + 1)

        for d in (0, 1):
            for m in range(4):
                b2_acc(d, m)
            for m in (4, 5):
                B2[(d, m)] = b2_start(d, m)
        a2_step(0)
        for d in (0, 1):
            for m in (4, 5):
                b2_acc(d, m)
            B2[(d, 6)] = b2_start(d, 6)
        a2_step(1)
        for d in (0, 1):
            b2_acc(d, 6)
        a2_step(2)

        out_ref[:, :AC] = accA_ref[pl.ds(g * SC + z * CH, CH), :]
        out_ref[:, AC:] = accB_ref[pl.ds(z * ZB + g * CH, CH), :]

        for rdma in (
            list(A1.values()) + list(B1.values())
            + list(A2.values()) + list(B2.values())
        ):
            rdma.wait_send()

    return pl.pallas_call(
        body,
        out_shape=jax.ShapeDtypeStruct((CH, N), jnp.float32),
        in_specs=[
            pl.BlockSpec(memory_space=pltpu.VMEM),
            pl.BlockSpec(memory_space=pltpu.VMEM),
        ],
        out_specs=pl.BlockSpec(memory_space=pltpu.VMEM),
        scratch_shapes=[
            pltpu.VMEM((M, M // N_DEV), jnp.float32),
            pltpu.VMEM((M, AC), jnp.float32),
            pltpu.VMEM((M, BC), jnp.float32),
            pltpu.VMEM((2, P - 1, SC, AH), jnp.float32),
            pltpu.VMEM((2, Z - 1, ZB, BH), jnp.float32),
            pltpu.VMEM((2, Z - 1, CH, AH), jnp.float32),
            pltpu.VMEM((2, P - 1, CH, BH), jnp.float32),
            pltpu.SemaphoreType.DMA((2, P - 1)),
            pltpu.SemaphoreType.DMA((2, P - 1)),
            pltpu.SemaphoreType.DMA((2, Z - 1)),
            pltpu.SemaphoreType.DMA((2, Z - 1)),
            pltpu.SemaphoreType.DMA((2, Z - 1)),
            pltpu.SemaphoreType.DMA((2, Z - 1)),
            pltpu.SemaphoreType.DMA((2, P - 1)),
            pltpu.SemaphoreType.DMA((2, P - 1)),
        ],
        compiler_params=pltpu.CompilerParams(
            collective_id=0, vmem_limit_bytes=64 * 1024 * 1024
        ),
    )(x, w_mat)


# device time: 89758 ns/iter; 2.1579x vs baseline; 1.0736x over previous
import jax
import jax.numpy as jnp
from jax import lax
from jax.experimental import pallas as pl
from jax.experimental.pallas import tpu as pltpu

N_DEV = 32
P = 8
Z = 4
M = 2048
N = 2048
SC = M // P
CH = M // N_DEV
ZB = M // Z

AC = 1280
BC = N - AC
AH = AC // 2
BH = BC // 2


def kernel(x, w_mat):
    def body(x_ref, w_ref, out_ref, xp_ref, accA_ref, accB_ref,
             cA1a, cA1b, cB1, cA2, cB2,
             sA1, rA1, sB1, rB1, sA2, rA2, sB2, rB2):
        my = lax.axis_index("i")
        g = lax.rem(my, P)
        plane0 = my - g
        z = lax.div(my, P)

        PI = (0, 1, 2, 5, 6, 7, 4, 3)
        INV = (0, 1, 2, 7, 6, 3, 4, 5)
        NEXT = (1, 2, 5, 0, 3, 6, 7, 4)
        PREV = (3, 0, 1, 4, 7, 2, 5, 6)

        def lut(table, idx):
            v = jnp.int32(table[0])
            for k in range(1, len(table)):
                v = jnp.where(idx == k, jnp.int32(table[k]), v)
            return v

        q = lut(INV, g)

        nbrs = [
            plane0 + lut(NEXT, g),
            plane0 + lut(PREV, g),
            lax.rem(my + P, N_DEV),
            lax.rem(my + N_DEV - P, N_DEV),
        ]

        barrier_sem = pltpu.get_barrier_semaphore()
        for nbr in nbrs:
            pl.semaphore_signal(
                barrier_sem, inc=1,
                device_id=(nbr,), device_id_type=pl.DeviceIdType.MESH,
            )
        pl.semaphore_wait(barrier_sem, len(nbrs))

        for gb in range(P):
            for zb in range(Z):
                xp_ref[pl.ds(gb * SC + zb * CH, CH), :] = (
                    x_ref[pl.ds(zb * ZB + gb * CH, CH), :]
                )

        def gemmA(pos):
            rows = pl.ds(lut(PI, lax.rem(pos + P, P)) * SC, SC)
            accA_ref[rows, :] = jnp.dot(
                xp_ref[rows, :], w_ref[:, :AC],
                preferred_element_type=jnp.float32,
            )

        def gemmB(zeta):
            rows = pl.ds(lax.rem(zeta + Z, Z) * ZB, ZB)
            accB_ref[rows, :] = jnp.dot(
                x_ref[rows, :], w_ref[:, AC:],
                preferred_element_type=jnp.float32,
            )

        def ip_send(s):
            return (lax.rem(q + (P - 1 - s), P), lax.rem(q + s + 1, P))

        def ip_recv(s):
            return (lax.rem(q + (P - 2 - s), P), lax.rem(q + s + 2, P))

        def z_send(s):
            return (lax.rem(z + (Z - 1 - s), Z), lax.rem(z + s + 1, Z))

        def z_recv(s):
            return (lax.rem(z + (Z - 2 - s), Z), lax.rem(z + s + 2, Z))

        def start(src, comm, ssem, rsem, d, s, to):
            rdma = pltpu.make_async_remote_copy(
                src_ref=src,
                dst_ref=comm.at[d, s],
                send_sem=ssem.at[d, s],
                recv_sem=rsem.at[d, s],
                device_id=(to,),
                device_id_type=pl.DeviceIdType.MESH,
            )
            rdma.start()
            return rdma

        AW = (384, AH - 384)

        def a1_cols(d, j):
            return pl.ds(d * AH + j * AW[0], AW[j])

        def a1_start(d, j, s):
            rows = pl.ds(lut(PI, ip_send(s)[d]) * SC, SC)
            src = accA_ref.at[rows, a1_cols(d, j)]
            comm = cA1a if j == 0 else cA1b
            rdma = pltpu.make_async_remote_copy(
                src_ref=src,
                dst_ref=comm.at[d, s],
                send_sem=sA1.at[d, j, s],
                recv_sem=rA1.at[d, j, s],
                device_id=(nbrs[d],),
                device_id_type=pl.DeviceIdType.MESH,
            )
            rdma.start()
            return rdma

        def a2_start(d, s):
            rows = pl.ds(g * SC + z_send(s)[d] * CH, CH)
            src = accA_ref.at[rows, pl.ds(d * AH, AH)]
            return start(src, cA2, sA2, rA2, d, s, nbrs[2 + d])

        def b1_start(d, s):
            rows = pl.ds(z_send(s)[d] * ZB, ZB)
            src = accB_ref.at[rows, pl.ds(d * BH, BH)]
            return start(src, cB1, sB1, rB1, d, s, nbrs[2 + d])

        GX = (0, 1, 1, 0, 0, 1, 1, 0)
        GY = (0, 0, 1, 1, 2, 2, 3, 3)
        PG1 = (1, 0, 3, 2, 5, 4, 7, 6)
        PG2 = (3, 2, 1, 0, 7, 6, 5, 4)
        PG3 = (4, 5, 6, 7, 0, 1, 2, 3)
        S0, S1 = (0, 3, 4, 7), (1, 2, 5, 6)
        T = (((0, 4), (3, 7)), ((1, 5), (2, 6)))
        gx = lut(GX, g)
        gyp = lax.rem(lut(GY, g), 2)

        def pick(m, tab0, tab1):
            return jnp.where(gx == 0, jnp.int32(tab0[m]), jnp.int32(tab1[m]))

        def pick2(j, parity):
            return jnp.where(
                gx == 0,
                jnp.where(parity == 0, jnp.int32(T[0][0][j]), jnp.int32(T[0][1][j])),
                jnp.where(parity == 0, jnp.int32(T[1][0][j]), jnp.int32(T[1][1][j])),
            )

        def b2_chunk(m, recv):
            if m < 4:
                return pick(m, S0, S1) if recv else pick(m, S1, S0)
            if m < 6:
                par = gyp if recv else 1 - gyp
                return pick2(m - 4, par)
            return g if recv else lut(PG3, g)

        def b2_partner(m):
            tab = PG1 if m < 4 else (PG2 if m < 6 else PG3)
            return plane0 + lut(tab, g)

        def b2_start(d, m):
            rows = pl.ds(z * ZB + b2_chunk(m, recv=False) * CH, CH)
            src = accB_ref.at[rows, pl.ds(d * BH, BH)]
            return start(src, cB2, sB2, rB2, d, m, b2_partner(m))

        def b2_acc(d, m):
            B2[(d, m)].wait_recv()
            rows = pl.ds(z * ZB + b2_chunk(m, recv=True) * CH, CH)
            colsd = pl.ds(d * BH, BH)
            accB_ref[rows, colsd] = accB_ref[rows, colsd] + cB2[d, m, :, :]

        gemmB(z + 3)
        gemmB(z + 1)
        B1 = {(d, 0): b1_start(d, 0) for d in (0, 1)}
        gemmA(q - 1)
        gemmA(q + 1)
        A1 = {(d, j, 0): a1_start(d, j, 0) for d in (0, 1) for j in (0, 1)}
        gemmB(z + 2)
        gemmB(z)
        gemmA(q - 2)
        gemmA(q + 2)
        gemmA(q - 3)
        gemmA(q + 3)
        gemmA(q + 4)
        gemmA(q)

        A2 = {}
        B2 = {}

        for s in range(P - 1):
            for j in (0, 1):
                for d in (0, 1):
                    A1[(d, j, s)].wait_recv()
                    rows = pl.ds(lut(PI, ip_recv(s)[d]) * SC, SC)
                    colsd = a1_cols(d, j)
                    comm = cA1a if j == 0 else cA1b
                    accA_ref[rows, colsd] = (
                        accA_ref[rows, colsd] + comm[d, s, :, :]
                    )
                    if s < P - 2:
                        A1[(d, j, s + 1)] = a1_start(d, j, s + 1)
                    elif j == 1:
                        A2[(d, 0)] = a2_start(d, 0)
            if s in (2, 4, 6):
                k = s // 2 - 1
                for d in (0, 1):
                    B1[(d, k)].wait_recv()
                    rows = pl.ds(z_recv(k)[d] * ZB, ZB)
                    colsd = pl.ds(d * BH, BH)
                    accB_ref[rows, colsd] = (
                        accB_ref[rows, colsd] + cB1[d, k, :, :]
                    )
                    if k < Z - 2:
                        B1[(d, k + 1)] = b1_start(d, k + 1)
                    else:
                        for m in range(4):
                            B2[(d, m)] = b2_start(d, m)

        def a2_step(k):
            for d in (0, 1):
                A2[(d, k)].wait_recv()
                rows = pl.ds(g * SC + z_recv(k)[d] * CH, CH)
                colsd = pl.ds(d * AH, AH)
                accA_ref[rows, colsd] = (
                    accA_ref[rows, colsd] + cA2[d, k, :, :]
                )
                if k < Z - 2:
                    A2[(d, k + 1)] = a2_start(d, k + 1)

        for d in (0, 1):
            for m in range(4):
                b2_acc(d, m)
            for m in (4, 5):
                B2[(d, m)] = b2_start(d, m)
        a2_step(0)
        for d in (0, 1):
            for m in (4, 5):
                b2_acc(d, m)
            B2[(d, 6)] = b2_start(d, 6)
        a2_step(1)
        for d in (0, 1):
            b2_acc(d, 6)
        a2_step(2)

        out_ref[:, :AC] = accA_ref[pl.ds(g * SC + z * CH, CH), :]
        out_ref[:, AC:] = accB_ref[pl.ds(z * ZB + g * CH, CH), :]

        for rdma in (
            list(A1.values()) + list(B1.values())
            + list(A2.values()) + list(B2.values())
        ):
            rdma.wait_send()

    return pl.pallas_call(
        body,
        out_shape=jax.ShapeDtypeStruct((CH, N), jnp.float32),
        in_specs=[
            pl.BlockSpec(memory_space=pltpu.VMEM),
            pl.BlockSpec(memory_space=pltpu.VMEM),
        ],
        out_specs=pl.BlockSpec(memory_space=pltpu.VMEM),
        scratch_shapes=[
            pltpu.VMEM((M, M // N_DEV), jnp.float32),
            pltpu.VMEM((M, AC), jnp.float32),
            pltpu.VMEM((M, BC), jnp.float32),
            pltpu.VMEM((2, P - 1, SC, 384), jnp.float32),
            pltpu.VMEM((2, P - 1, SC, AH - 384), jnp.float32),
            pltpu.VMEM((2, Z - 1, ZB, BH), jnp.float32),
            pltpu.VMEM((2, Z - 1, CH, AH), jnp.float32),
            pltpu.VMEM((2, P - 1, CH, BH), jnp.float32),
            pltpu.SemaphoreType.DMA((2, 2, P - 1)),
            pltpu.SemaphoreType.DMA((2, 2, P - 1)),
            pltpu.SemaphoreType.DMA((2, Z - 1)),
            pltpu.SemaphoreType.DMA((2, Z - 1)),
            pltpu.SemaphoreType.DMA((2, Z - 1)),
            pltpu.SemaphoreType.DMA((2, Z - 1)),
            pltpu.SemaphoreType.DMA((2, P - 1)),
            pltpu.SemaphoreType.DMA((2, P - 1)),
        ],
        compiler_params=pltpu.CompilerParams(
            collective_id=0, vmem_limit_bytes=64 * 1024 * 1024
        ),
    )(x, w_mat)
